# baseline (speedup 1.0000x reference)
import dataclasses

import jax
import jax.numpy as jnp
from jax import lax
from jax.experimental import pallas as pl
from jax.experimental.pallas import tpu as pltpu
from jax.experimental.pallas import tpu_sc as plsc

B = 16384
F = 16
NC = 2
NS = 16
NW = NC * NS
BPW = B // NW
CH = 128
NCH = BPW // CH
GROUPS = BPW // F


def _fm_body(uix, iix, aix, bix, ue, ie, ae, be, lu, li, la, lb, out,
             uiv, iiv, aiv, biv, ur, ir, ar, br, lur, lir, lar, lbr,
             dtmp, ov, sem):
    wid = lax.axis_index("s") * NC + lax.axis_index("c")
    base = wid * BPW

    idx_copies = [
        pltpu.async_copy(uix.at[wid], uiv, sem),
        pltpu.async_copy(iix.at[wid], iiv, sem),
        pltpu.async_copy(aix.at[wid], aiv, sem),
        pltpu.async_copy(bix.at[wid], biv, sem),
    ]
    for c in idx_copies:
        c.wait()

    copies = []
    for j in range(NCH):
        sl = pl.ds(j * CH, CH)
        copies.append(pltpu.async_copy(ue.at[uiv.at[j]], ur.at[sl], sem))
        copies.append(pltpu.async_copy(ie.at[iiv.at[j]], ir.at[sl], sem))
        copies.append(pltpu.async_copy(ae.at[aiv.at[j]], ar.at[sl], sem))
        copies.append(pltpu.async_copy(be.at[biv.at[j]], br.at[sl], sem))
        copies.append(pltpu.async_copy(lu.at[uiv.at[j]], lur.at[sl], sem))
        copies.append(pltpu.async_copy(li.at[iiv.at[j]], lir.at[sl], sem))
        copies.append(pltpu.async_copy(la.at[aiv.at[j]], lar.at[sl], sem))
        copies.append(pltpu.async_copy(lb.at[biv.at[j]], lbr.at[sl], sem))
    for c in copies:
        c.wait()

    lane = lax.broadcasted_iota(jnp.int32, (F,), 0)

    @pl.loop(0, GROUPS)
    def _(g):
        row0 = g * F
        for r in range(F):
            u = ur[row0 + r, :]
            i = ir[row0 + r, :]
            a = ar[row0 + r, :]
            b = br[row0 + r, :]
            s = u + i + a + b
            dtmp[r, :] = s * s - u * u - i * i - a * a - b * b
        pw = jnp.zeros((F,), jnp.float32)
        for j in range(F):
            pw = pw + plsc.load_gather(dtmp, [lane, jnp.full((F,), j, jnp.int32)])
        acc = (lur[pl.ds(row0, F)] + lir[pl.ds(row0, F)]
               + lar[pl.ds(row0, F)] + lbr[pl.ds(row0, F)])
        tot = acc + 0.5 * pw
        ov[pl.ds(row0, F)] = 1.0 / (1.0 + jnp.exp(-tot))

    pltpu.sync_copy(ov, out.at[pl.ds(base, BPW)])


def kernel(user, item, metadata, user_emb, item_emb, meta_emb0, meta_emb1,
           lin_user, lin_item, lin_meta0, lin_meta1):
    mesh = plsc.VectorSubcoreMesh(core_axis_name="c", subcore_axis_name="s")
    cp = pltpu.CompilerParams()
    fields = pltpu.CompilerParams.__dataclass_fields__
    if "needs_layout_passes" in fields:
        cp = dataclasses.replace(cp, needs_layout_passes=False)
    if "use_tc_tiling_on_sc" in fields:
        cp = dataclasses.replace(cp, use_tc_tiling_on_sc=False)
    fm = pl.kernel(
        _fm_body,
        out_type=jax.ShapeDtypeStruct((B,), jnp.float32),
        mesh=mesh,
        compiler_params=cp,
        scratch_types=[
            pltpu.VMEM((NCH, CH), jnp.int32),
            pltpu.VMEM((NCH, CH), jnp.int32),
            pltpu.VMEM((NCH, CH), jnp.int32),
            pltpu.VMEM((NCH, CH), jnp.int32),
            pltpu.VMEM((BPW, F), jnp.float32),
            pltpu.VMEM((BPW, F), jnp.float32),
            pltpu.VMEM((BPW, F), jnp.float32),
            pltpu.VMEM((BPW, F), jnp.float32),
            pltpu.VMEM((BPW,), jnp.float32),
            pltpu.VMEM((BPW,), jnp.float32),
            pltpu.VMEM((BPW,), jnp.float32),
            pltpu.VMEM((BPW,), jnp.float32),
            pltpu.VMEM((F, F), jnp.float32),
            pltpu.VMEM((BPW,), jnp.float32),
            pltpu.SemaphoreType.DMA,
        ],
    )
    uix = user.astype(jnp.int32).reshape(NW, NCH, CH)
    iix = item.astype(jnp.int32).reshape(NW, NCH, CH)
    aix = metadata[:, 0].astype(jnp.int32).reshape(NW, NCH, CH)
    bix = metadata[:, 1].astype(jnp.int32).reshape(NW, NCH, CH)
    return fm(uix, iix, aix, bix, user_emb, item_emb, meta_emb0, meta_emb1,
              lin_user.reshape(-1), lin_item.reshape(-1),
              lin_meta0.reshape(-1), lin_meta1.reshape(-1))
